# baseline (speedup 1.0000x reference)
import jax
import jax.numpy as jnp
from jax import lax
from jax.experimental import pallas as pl
from jax.experimental.pallas import tpu as pltpu
from jax.experimental.pallas import tpu_sc as plsc

N = 10000
IN = 256
H = 4
D = 64
E = 160000

NPAD = 10240
RNG = 320
K = 8192
C = 1600
G = 64
BLK = 1000


def _mm_kernel(h_ref, w_ref, m_ref, f0_ref, f1_ref, ee_ref):
    f = jnp.dot(h_ref[...], w_ref[...], preferred_element_type=jnp.float32)
    f0_ref[...] = f[:, :IN]
    f1_ref[...] = f[:, IN:]
    ee_ref[...] = jnp.dot(f, m_ref[...], preferred_element_type=jnp.float32)


def _feat_and_ee(h, W0, W1, al0, ar0, al1, ar1):
    W = jnp.concatenate([W0, W1], axis=1)
    rows = jnp.arange(H * D)
    heads = rows // D
    M = jnp.zeros((2 * IN, 16), jnp.float32)
    M = M.at[rows, heads].set(al0.reshape(H * D))
    M = M.at[IN + rows, 4 + heads].set(al1.reshape(H * D))
    M = M.at[rows, 8 + heads].set(ar0.reshape(H * D))
    M = M.at[IN + rows, 12 + heads].set(ar1.reshape(H * D))
    f0, f1, ee = pl.pallas_call(
        _mm_kernel,
        grid=(N // BLK,),
        in_specs=[
            pl.BlockSpec((BLK, IN), lambda i: (i, 0)),
            pl.BlockSpec((IN, 2 * IN), lambda i: (0, 0)),
            pl.BlockSpec((2 * IN, 16), lambda i: (0, 0)),
        ],
        out_specs=[
            pl.BlockSpec((BLK, IN), lambda i: (i, 0)),
            pl.BlockSpec((BLK, IN), lambda i: (i, 0)),
            pl.BlockSpec((BLK, 16), lambda i: (i, 0)),
        ],
        out_shape=[
            jax.ShapeDtypeStruct((N, IN), jnp.float32),
            jax.ShapeDtypeStruct((N, IN), jnp.float32),
            jax.ShapeDtypeStruct((N, 16), jnp.float32),
        ],
    )(h, W, M)
    return f0, f1, ee


def _sc_body(pk0, pk1, eeP, f0, f1,
             acc0, acc1, den0, den1,
             accum, denom, eem, klist, pkchunk, gidx, fstage, estage, sbuf,
             sem1, sem2):
    cidx = lax.axis_index("c")
    sidx = lax.axis_index("s")
    w = sidx * 2 + cidx
    i16 = lax.iota(jnp.int32, 16)
    zf16 = jnp.zeros((16,), jnp.float32)

    for k in range(10):
        pltpu.sync_copy(eeP.at[pl.ds(w * 32 + 1024 * k, 32)],
                        eem.at[pl.ds(32 * k, 32)])

    def zero_acc(r, _):
        for t in range(16):
            accum[r, pl.ds(t * 16, 16)] = zf16
        denom[r] = zf16
        return 0

    def compact(pk_ref):
        def inner(j, n):
            n = jnp.minimum(n, K - 16)
            pkv = pkchunk[pl.ds(j * 16, 16)]
            dv = pkv >> 14
            msk = ((dv >> 5) & 31) == w
            dl = (dv & 31) + ((dv >> 10) << 5)
            klv = (dl << 14) | (pkv & 16383)
            plsc.store_compressed(klist.at[pl.ds(n, 16)], klv, mask=msk)
            return n + plsc.all_reduce_population_count(msk)[0]

        def outer(t, n):
            pltpu.sync_copy(pk_ref.at[pl.ds(t * C, C)], pkchunk)
            return lax.fori_loop(0, C // 16, inner, n, unroll=4)

        n = lax.fori_loop(0, E // C, outer, jnp.int32(0))
        n = jnp.minimum(n, K - G - 16)
        for q in range(4):
            posq = n + q * 16 + i16
            plsc.store_scatter(klist, [posq],
                               jnp.zeros((16,), jnp.int32), mask=posq < K)
        return n

    def accumulate(m, n, f_ref):
        nblk = (n + (G - 1)) >> 6

        def gbody(g, _):
            base = g * G

            def unpack(j, _):
                kv = klist[pl.ds(base + j * 16, 16)]
                gidx[pl.ds(j * 16, 16)] = kv & 16383
                return 0

            lax.fori_loop(0, G // 16, unpack, 0)
            cp1 = pltpu.async_copy(f_ref.at[gidx], fstage, sem1)
            cp2 = pltpu.async_copy(eeP.at[gidx], estage, sem2)
            cp1.wait()
            cp2.wait()

            def process(j, _):
                kv = klist[pl.ds(base + j * 16, 16)]
                dl = kv >> 14
                rows = j * 16 + i16
                valid = rows < (n - base)
                svals = []
                for h in range(H):
                    col_l = jnp.full((16,), m * 4 + h, jnp.int32)
                    elh = plsc.load_gather(estage, [rows, col_l])
                    erh = plsc.load_gather(
                        eem, [dl, jnp.full((16,), 8 + m * 4 + h, jnp.int32)])
                    e = elh + erh
                    e = jnp.where(e > 0, e, 0.2 * e)
                    sh = jnp.where(valid, jnp.exp(e), 0.0)
                    svals.append(sh)
                    plsc.store_scatter(sbuf, [rows * 16 + h], sh)
                for l in range(16):
                    dls = dl[l]
                    le = j * 16 + l
                    sv = sbuf[pl.ds(le * 16, 16)]
                    plsc.addupdate(denom.at[dls], sv)
                    loads = [fstage[le, pl.ds(c * 16, 16)] for c in range(16)]
                    prods = [svals[c // 4][l] * loads[c] for c in range(16)]
                    for c in range(16):
                        plsc.addupdate(accum.at[dls, pl.ds(c * 16, 16)],
                                       prods[c])
                return 0

            lax.fori_loop(0, G // 16, process, 0, unroll=2)
            return 0

        lax.fori_loop(0, nblk, gbody, 0)

    def run_metapath(pk_ref, f_ref, m, acc_out, den_out):
        lax.fori_loop(0, RNG, zero_acc, 0)
        n = compact(pk_ref)
        accumulate(m, n, f_ref)
        for k in range(10):
            pltpu.sync_copy(accum.at[pl.ds(32 * k, 32)],
                            acc_out.at[pl.ds(w * 32 + 1024 * k, 32)])
            pltpu.sync_copy(denom.at[pl.ds(32 * k, 32)],
                            den_out.at[pl.ds(w * 32 + 1024 * k, 32)])

    run_metapath(pk0, f0, 0, acc0, den0)
    run_metapath(pk1, f1, 1, acc1, den1)


def _sc_gat(pk0, pk1, eeP, f0, f1):
    mesh = plsc.VectorSubcoreMesh(core_axis_name="c", subcore_axis_name="s",
                                  num_cores=2, num_subcores=16)
    return pl.kernel(
        _sc_body,
        out_type=[
            jax.ShapeDtypeStruct((NPAD, IN), jnp.float32),
            jax.ShapeDtypeStruct((NPAD, IN), jnp.float32),
            jax.ShapeDtypeStruct((NPAD, 16), jnp.float32),
            jax.ShapeDtypeStruct((NPAD, 16), jnp.float32),
        ],
        mesh=mesh,
        compiler_params=pltpu.CompilerParams(needs_layout_passes=False, use_tc_tiling_on_sc=False),
        scratch_types=[
            pltpu.VMEM((RNG, IN), jnp.float32),
            pltpu.VMEM((RNG, 16), jnp.float32),
            pltpu.VMEM((RNG, 16), jnp.float32),
            pltpu.VMEM((K,), jnp.int32),
            pltpu.VMEM((C,), jnp.int32),
            pltpu.VMEM((G,), jnp.int32),
            pltpu.VMEM((G, IN), jnp.float32),
            pltpu.VMEM((G, 16), jnp.float32),
            pltpu.VMEM((G * 16,), jnp.float32),
            pltpu.SemaphoreType.DMA,
            pltpu.SemaphoreType.DMA,
        ],
    )(pk0, pk1, eeP, f0, f1)


def _post_kernel(a0_ref, a1_ref, d0_ref, d1_ref, b_ref, o_ref):
    bpk = BLK
    b = b_ref[...]
    def side(a, d, m):
        cols = [jnp.broadcast_to(d[:, h:h + 1], (bpk, D))
                for h in range(H)]
        dd = jnp.concatenate(cols, axis=1) + 1e-9
        x = a / dd + b[m:m + 1, :]
        return jnp.where(x > 0, x, jnp.exp(jnp.minimum(x, 0.0)) - 1.0)
    x0 = side(a0_ref[...], d0_ref[...], 0)
    x1 = side(a1_ref[...], d1_ref[...], 1)
    o_ref[...] = jnp.maximum(x0, x1)


def _post(acc0, acc1, den0, den1, b0, b1):
    bias = jnp.stack([b0, b1])
    return pl.pallas_call(
        _post_kernel,
        grid=(N // BLK,),
        in_specs=[
            pl.BlockSpec((BLK, IN), lambda i: (i, 0)),
            pl.BlockSpec((BLK, IN), lambda i: (i, 0)),
            pl.BlockSpec((BLK, 16), lambda i: (i, 0)),
            pl.BlockSpec((BLK, 16), lambda i: (i, 0)),
            pl.BlockSpec((2, IN), lambda i: (0, 0)),
        ],
        out_specs=pl.BlockSpec((BLK, IN), lambda i: (i, 0)),
        out_shape=jax.ShapeDtypeStruct((N, IN), jnp.float32),
    )(acc0, acc1, den0, den1, bias)


def kernel(h, ei0, ei1, W0, al0, ar0, b0, W1, al1, ar1, b1):
    f0, f1, ee = _feat_and_ee(h, W0, W1, al0, ar0, al1, ar1)
    eeP = jnp.pad(ee, ((0, NPAD - N), (0, 0)))
    pk0 = (ei0[1] << 14) | ei0[0]
    pk1 = (ei1[1] << 14) | ei1[0]
    acc0, acc1, den0, den1 = _sc_gat(pk0, pk1, eeP, f0, f1)
    return _post(acc0, acc1, den0, den1, b0, b1)


# speedup vs baseline: 1.0428x; 1.0428x over previous
import jax
import jax.numpy as jnp
from jax import lax
from jax.experimental import pallas as pl
from jax.experimental.pallas import tpu as pltpu
from jax.experimental.pallas import tpu_sc as plsc

N = 10000
IN = 256
H = 4
D = 64
E = 160000

NPAD = 10240
RNG = 320
K = 8192
C = 1600
G = 64
BLK = 1000


def _mm_kernel(h_ref, w_ref, m_ref, f0_ref, f1_ref, ee_ref):
    f = jnp.dot(h_ref[...], w_ref[...], preferred_element_type=jnp.float32)
    f0_ref[...] = f[:, :IN]
    f1_ref[...] = f[:, IN:]
    ee_ref[...] = jnp.dot(f, m_ref[...], preferred_element_type=jnp.float32)


def _feat_and_ee(h, W0, W1, al0, ar0, al1, ar1):
    W = jnp.concatenate([W0, W1], axis=1)
    rows = jnp.arange(H * D)
    heads = rows // D
    M = jnp.zeros((2 * IN, 16), jnp.float32)
    M = M.at[rows, heads].set(al0.reshape(H * D))
    M = M.at[IN + rows, 4 + heads].set(al1.reshape(H * D))
    M = M.at[rows, 8 + heads].set(ar0.reshape(H * D))
    M = M.at[IN + rows, 12 + heads].set(ar1.reshape(H * D))
    f0, f1, ee = pl.pallas_call(
        _mm_kernel,
        grid=(N // BLK,),
        in_specs=[
            pl.BlockSpec((BLK, IN), lambda i: (i, 0)),
            pl.BlockSpec((IN, 2 * IN), lambda i: (0, 0)),
            pl.BlockSpec((2 * IN, 16), lambda i: (0, 0)),
        ],
        out_specs=[
            pl.BlockSpec((BLK, IN), lambda i: (i, 0)),
            pl.BlockSpec((BLK, IN), lambda i: (i, 0)),
            pl.BlockSpec((BLK, 16), lambda i: (i, 0)),
        ],
        out_shape=[
            jax.ShapeDtypeStruct((N, IN), jnp.float32),
            jax.ShapeDtypeStruct((N, IN), jnp.float32),
            jax.ShapeDtypeStruct((N, 16), jnp.float32),
        ],
    )(h, W, M)
    return f0, f1, ee


def _sc_body(pk0, pk1, eeP, f0, f1,
             acc0, acc1, den0, den1,
             accum, denom, eem, klist, pkchunk, gidx0,
             fstage0, estage0, sbuf,
             sem1, sem2):
    cidx = lax.axis_index("c")
    sidx = lax.axis_index("s")
    w = sidx * 2 + cidx
    i16 = lax.iota(jnp.int32, 16)
    zf16 = jnp.zeros((16,), jnp.float32)

    for k in range(10):
        pltpu.sync_copy(eeP.at[pl.ds(w * 32 + 1024 * k, 32), pl.ds(8, 8)],
                        eem.at[pl.ds(32 * k, 32)])

    def zero_acc(r, _):
        for t in range(16):
            accum[r, pl.ds(t * 16, 16)] = zf16
        denom[r] = zf16
        return 0

    def compact(pk_ref):
        def inner(j, n):
            n = jnp.minimum(n, K - 16)
            pkv = pkchunk[pl.ds(j * 16, 16)]
            dv = pkv >> 14
            msk = ((dv >> 5) & 31) == w
            dl = (dv & 31) + ((dv >> 10) << 5)
            klv = (dl << 14) | (pkv & 16383)
            plsc.store_compressed(klist.at[pl.ds(n, 16)], klv, mask=msk)
            return n + plsc.all_reduce_population_count(msk)[0]

        def outer(t, n):
            pltpu.sync_copy(pk_ref.at[pl.ds(t * C, C)], pkchunk)
            return lax.fori_loop(0, C // 16, inner, n)

        n = lax.fori_loop(0, E // C, outer, jnp.int32(0))
        n = jnp.minimum(n, K - G - 16)
        for q in range(4):
            posq = n + q * 16 + i16
            plsc.store_scatter(klist, [posq],
                               jnp.zeros((16,), jnp.int32), mask=posq < K)
        return n

    def accumulate(m, n, f_ref):
        nblk = (n + (G - 1)) >> 6

        def gbody(g, _):
            base = g * G

            def unpack(j, _):
                kv = klist[pl.ds(base + j * 16, 16)]
                gidx0[pl.ds(j * 16, 16)] = kv & 16383
                return 0

            lax.fori_loop(0, G // 16, unpack, 0)
            cp1 = pltpu.async_copy(f_ref.at[gidx0], fstage0, sem1)
            cp2 = pltpu.async_copy(eeP.at[gidx0], estage0, sem2)
            cp1.wait()
            cp2.wait()

            def process(j, _):
                kv = klist[pl.ds(base + j * 16, 16)]
                dl = kv >> 14
                rows = j * 16 + i16
                valid = rows < (n - base)
                svals = []
                for h in range(H):
                    col_l = jnp.full((16,), m * 4 + h, jnp.int32)
                    elh = plsc.load_gather(estage0, [rows, col_l])
                    erh = plsc.load_gather(
                        eem, [dl, jnp.full((16,), m * 4 + h, jnp.int32)])
                    e = elh + erh
                    e = jnp.where(e > 0, e, 0.2 * e)
                    sh = jnp.where(valid, jnp.exp(e), 0.0)
                    svals.append(sh)
                    plsc.addupdate_scatter(
                        denom, [dl, jnp.full((16,), h, jnp.int32)], sh)
                for l in range(16):
                    dls = dl[l]
                    le = j * 16 + l
                    loads = [fstage0[le, pl.ds(c * 16, 16)] for c in range(16)]
                    prods = [svals[c // 4][l] * loads[c] for c in range(16)]
                    for c in range(16):
                        plsc.addupdate(accum.at[dls, pl.ds(c * 16, 16)],
                                       prods[c])
                return 0

            lax.fori_loop(0, G // 16, process, 0)
            return 0

        lax.fori_loop(0, nblk, gbody, 0)

    def run_metapath(pk_ref, f_ref, m, acc_out, den_out):
        lax.fori_loop(0, RNG, zero_acc, 0)
        n = compact(pk_ref)
        accumulate(m, n, f_ref)
        for k in range(10):
            pltpu.sync_copy(accum.at[pl.ds(32 * k, 32)],
                            acc_out.at[pl.ds(w * 32 + 1024 * k, 32)])
            pltpu.sync_copy(denom.at[pl.ds(32 * k, 32)],
                            den_out.at[pl.ds(w * 32 + 1024 * k, 32)])

    run_metapath(pk0, f0, 0, acc0, den0)
    run_metapath(pk1, f1, 1, acc1, den1)


def _sc_gat(pk0, pk1, eeP, f0, f1):
    mesh = plsc.VectorSubcoreMesh(core_axis_name="c", subcore_axis_name="s",
                                  num_cores=2, num_subcores=16)
    return pl.kernel(
        _sc_body,
        out_type=[
            jax.ShapeDtypeStruct((NPAD, IN), jnp.float32),
            jax.ShapeDtypeStruct((NPAD, IN), jnp.float32),
            jax.ShapeDtypeStruct((NPAD, 16), jnp.float32),
            jax.ShapeDtypeStruct((NPAD, 16), jnp.float32),
        ],
        mesh=mesh,
        compiler_params=pltpu.CompilerParams(needs_layout_passes=False, use_tc_tiling_on_sc=False),
        scratch_types=[
            pltpu.VMEM((RNG, IN), jnp.float32),
            pltpu.VMEM((RNG, 16), jnp.float32),
            pltpu.VMEM((RNG, 8), jnp.float32),
            pltpu.VMEM((K,), jnp.int32),
            pltpu.VMEM((C,), jnp.int32),
            pltpu.VMEM((G,), jnp.int32),
            pltpu.VMEM((G, IN), jnp.float32),
            pltpu.VMEM((G, 16), jnp.float32),
            pltpu.VMEM((G * 16,), jnp.float32),
            pltpu.SemaphoreType.DMA,
            pltpu.SemaphoreType.DMA,
        ],
    )(pk0, pk1, eeP, f0, f1)


def _post_kernel(a0_ref, a1_ref, d0_ref, d1_ref, b_ref, o_ref):
    bpk = BLK
    b = b_ref[...]
    def side(a, d, m):
        cols = [jnp.broadcast_to(d[:, h:h + 1], (bpk, D))
                for h in range(H)]
        dd = jnp.concatenate(cols, axis=1) + 1e-9
        x = a / dd + b[m:m + 1, :]
        return jnp.where(x > 0, x, jnp.exp(jnp.minimum(x, 0.0)) - 1.0)
    x0 = side(a0_ref[...], d0_ref[...], 0)
    x1 = side(a1_ref[...], d1_ref[...], 1)
    o_ref[...] = jnp.maximum(x0, x1)


def _post(acc0, acc1, den0, den1, b0, b1):
    bias = jnp.stack([b0, b1])
    return pl.pallas_call(
        _post_kernel,
        grid=(N // BLK,),
        in_specs=[
            pl.BlockSpec((BLK, IN), lambda i: (i, 0)),
            pl.BlockSpec((BLK, IN), lambda i: (i, 0)),
            pl.BlockSpec((BLK, 16), lambda i: (i, 0)),
            pl.BlockSpec((BLK, 16), lambda i: (i, 0)),
            pl.BlockSpec((2, IN), lambda i: (0, 0)),
        ],
        out_specs=pl.BlockSpec((BLK, IN), lambda i: (i, 0)),
        out_shape=jax.ShapeDtypeStruct((N, IN), jnp.float32),
    )(acc0, acc1, den0, den1, bias)


def kernel(h, ei0, ei1, W0, al0, ar0, b0, W1, al1, ar1, b1):
    f0, f1, ee = _feat_and_ee(h, W0, W1, al0, ar0, al1, ar1)
    eeP = jnp.pad(ee, ((0, NPAD - N), (0, 0)))
    pk0 = (ei0[1] << 14) | ei0[0]
    pk1 = (ei1[1] << 14) | ei1[0]
    acc0, acc1, den0, den1 = _sc_gat(pk0, pk1, eeP, f0, f1)
    return _post(acc0, acc1, den0, den1, b0, b1)
